# baseline (speedup 1.0000x reference)
import jax
import jax.numpy as jnp
from jax import lax
from jax.experimental import pallas as pl
from jax.experimental.pallas import tpu as pltpu

NSEG = 16
BLK = 8192
BPR = BLK // 128


def _treemax(t):
    r = t.shape[0]
    while r > 8:
        r //= 2
        t = jnp.maximum(t[:r], t[r:])
    return jnp.max(t, axis=0, keepdims=True)


def _pool_kernel(blo_ref, bhi_ref, x_ref, pos_ref, w1_ref, w2_ref, b_ref,
                 batch_ref, out_ref):
    i = pl.program_id(0)
    nblk = pl.num_programs(0)

    @pl.when(i == 0)
    def _init():
        out_ref[...] = jnp.full((NSEG, 64), -jnp.inf, dtype=jnp.float32)

    z = jnp.dot(x_ref[...], w1_ref[...], preferred_element_type=jnp.float32)
    z = z + jnp.dot(pos_ref[...], w2_ref[...], preferred_element_type=jnp.float32)

    lo = blo_ref[i]
    hi = bhi_ref[i]

    @pl.when(lo == hi)
    def _pure():
        v = _treemax(z)
        cur = out_ref[pl.ds(lo, 1), :]
        out_ref[pl.ds(lo, 1), :] = jnp.maximum(cur, v)

    @pl.when(lo != hi)
    def _mixed():
        bb = batch_ref[...]
        riota = lax.broadcasted_iota(jnp.int32, (BLK, 1), 0)
        for s in range(NSEG):
            @pl.when(jnp.logical_and(lo <= s, s <= hi))
            def _acc(s=s):
                start = jnp.sum((bb < s).astype(jnp.int32))
                end = jnp.sum((bb <= s).astype(jnp.int32))
                m = jnp.logical_and(riota >= start, riota < end)
                v = _treemax(jnp.where(m, z, -jnp.inf))
                cur = out_ref[pl.ds(s, 1), :]
                out_ref[pl.ds(s, 1), :] = jnp.maximum(cur, v)

    @pl.when(i == nblk - 1)
    def _finish():
        acc = out_ref[...]
        res = jnp.maximum(acc + b_ref[...], 0.0)
        out_ref[...] = jnp.where(acc == -jnp.inf, acc, res)


def kernel(x, pos, W, b, batch):
    n = x.shape[0]
    nblk = n // BLK

    w1 = W[:, :61].T
    w2 = W[:, 61:].T
    b2 = b.reshape(1, 64)
    batch = batch.astype(jnp.int32)
    batchv = batch.reshape(n // 128, 128)
    blo = batch[::BLK]
    bhi = batch[BLK - 1::BLK]

    grid_spec = pltpu.PrefetchScalarGridSpec(
        num_scalar_prefetch=2,
        grid=(nblk,),
        in_specs=[
            pl.BlockSpec((BLK, 61), lambda i, *_: (i, 0)),
            pl.BlockSpec((BLK, 3), lambda i, *_: (i, 0)),
            pl.BlockSpec((61, 64), lambda i, *_: (0, 0)),
            pl.BlockSpec((3, 64), lambda i, *_: (0, 0)),
            pl.BlockSpec((1, 64), lambda i, *_: (0, 0)),
            pl.BlockSpec((BPR, 128), lambda i, *_: (i, 0)),
        ],
        out_specs=pl.BlockSpec((NSEG, 64), lambda i, *_: (0, 0)),
    )

    return pl.pallas_call(
        _pool_kernel,
        grid_spec=grid_spec,
        out_shape=jax.ShapeDtypeStruct((NSEG, 64), jnp.float32),
    )(blo, bhi, x, pos, w1, w2, b2, batchv)


# speedup vs baseline: 1.0931x; 1.0931x over previous
import jax
import jax.numpy as jnp
from jax import lax
from jax.experimental import pallas as pl
from jax.experimental.pallas import tpu as pltpu

NSEG = 16
BLK = 4096
BPR = BLK // 128


def _treemax(t):
    r = t.shape[0]
    while r > 8:
        r //= 2
        t = jnp.maximum(t[:r], t[r:])
    return jnp.max(t, axis=0, keepdims=True)


def _pool_kernel(blo_ref, bhi_ref, x_ref, pos_ref, w1_ref, w2_ref, b_ref,
                 batch_ref, out_ref):
    i = pl.program_id(0)
    nblk = pl.num_programs(0)

    @pl.when(i == 0)
    def _init():
        out_ref[...] = jnp.full((NSEG, 64), -jnp.inf, dtype=jnp.float32)

    z = jnp.dot(x_ref[...], w1_ref[...], preferred_element_type=jnp.float32)
    z = z + jnp.dot(pos_ref[...], w2_ref[...], preferred_element_type=jnp.float32)

    lo = blo_ref[i]
    hi = bhi_ref[i]
    bb = batch_ref[...]
    riota = lax.broadcasted_iota(jnp.int32, (BLK, 1), 0)

    end_lo = jnp.sum((bb <= lo).astype(jnp.int32))
    vh = _treemax(jnp.where(riota < end_lo, z, -jnp.inf))
    cur = out_ref[pl.ds(lo, 1), :]
    out_ref[pl.ds(lo, 1), :] = jnp.maximum(cur, vh)

    start_hi = jnp.sum((bb < hi).astype(jnp.int32))
    vt = _treemax(jnp.where(riota >= start_hi, z, -jnp.inf))
    cur = out_ref[pl.ds(hi, 1), :]
    out_ref[pl.ds(hi, 1), :] = jnp.maximum(cur, vt)

    for s in range(1, NSEG - 1):
        @pl.when(jnp.logical_and(lo < s, s < hi))
        def _interior(s=s):
            start = jnp.sum((bb < s).astype(jnp.int32))
            end = jnp.sum((bb <= s).astype(jnp.int32))
            m = jnp.logical_and(riota >= start, riota < end)
            v = _treemax(jnp.where(m, z, -jnp.inf))
            cur2 = out_ref[pl.ds(s, 1), :]
            out_ref[pl.ds(s, 1), :] = jnp.maximum(cur2, v)

    @pl.when(i == nblk - 1)
    def _finish():
        acc = out_ref[...]
        res = jnp.maximum(acc + b_ref[...], 0.0)
        out_ref[...] = jnp.where(acc == -jnp.inf, acc, res)


def kernel(x, pos, W, b, batch):
    n = x.shape[0]
    nblk = n // BLK

    w1 = W[:, :61].T
    w2 = W[:, 61:].T
    b2 = b.reshape(1, 64)
    batch = batch.astype(jnp.int32)
    batchv = batch.reshape(n // 128, 128)
    blo = batch[::BLK]
    bhi = batch[BLK - 1::BLK]

    grid_spec = pltpu.PrefetchScalarGridSpec(
        num_scalar_prefetch=2,
        grid=(nblk,),
        in_specs=[
            pl.BlockSpec((BLK, 61), lambda i, *_: (i, 0)),
            pl.BlockSpec((BLK, 3), lambda i, *_: (i, 0)),
            pl.BlockSpec((61, 64), lambda i, *_: (0, 0)),
            pl.BlockSpec((3, 64), lambda i, *_: (0, 0)),
            pl.BlockSpec((1, 64), lambda i, *_: (0, 0)),
            pl.BlockSpec((BPR, 128), lambda i, *_: (i, 0)),
        ],
        out_specs=pl.BlockSpec((NSEG, 64), lambda i, *_: (0, 0)),
    )

    return pl.pallas_call(
        _pool_kernel,
        grid_spec=grid_spec,
        out_shape=jax.ShapeDtypeStruct((NSEG, 64), jnp.float32),
    )(blo, bhi, x, pos, w1, w2, b2, batchv)


# speedup vs baseline: 1.0973x; 1.0038x over previous
import jax
import jax.numpy as jnp
from jax import lax
from jax.experimental import pallas as pl
from jax.experimental.pallas import tpu as pltpu

NSEG = 16
BLK = 4096
BPR = BLK // 128


def _treemax(t):
    r = t.shape[0]
    while r > 8:
        r //= 2
        t = jnp.maximum(t[:r], t[r:])
    return jnp.max(t, axis=0, keepdims=True)


def _pool_kernel(blo_ref, bhi_ref, x_ref, pos_ref, w1_ref, w2_ref, b_ref,
                 batch_ref, out_ref):
    i = pl.program_id(0)
    nblk = pl.num_programs(0)

    @pl.when(i == 0)
    def _init():
        out_ref[...] = jnp.full((NSEG, 64), -jnp.inf, dtype=jnp.float32)

    z = jnp.dot(x_ref[...], w1_ref[...], preferred_element_type=jnp.float32)
    z = z + jnp.dot(pos_ref[...], w2_ref[...], preferred_element_type=jnp.float32)

    lo = blo_ref[i]
    hi = bhi_ref[i]
    bb = batch_ref[...]
    riota = lax.broadcasted_iota(jnp.int32, (BLK, 1), 0)

    end_lo = jnp.sum((bb <= lo).astype(jnp.int32))
    start_hi = jnp.sum((bb < hi).astype(jnp.int32))
    zz = jnp.concatenate(
        [jnp.where(riota < end_lo, z, -jnp.inf),
         jnp.where(riota >= start_hi, z, -jnp.inf)], axis=1)
    v2 = _treemax(zz)
    vh = v2[:, :64]
    vt = v2[:, 64:]
    cur = out_ref[pl.ds(lo, 1), :]
    out_ref[pl.ds(lo, 1), :] = jnp.maximum(cur, vh)
    cur = out_ref[pl.ds(hi, 1), :]
    out_ref[pl.ds(hi, 1), :] = jnp.maximum(cur, vt)

    for s in range(1, NSEG - 1):
        @pl.when(jnp.logical_and(lo < s, s < hi))
        def _interior(s=s):
            start = jnp.sum((bb < s).astype(jnp.int32))
            end = jnp.sum((bb <= s).astype(jnp.int32))
            m = jnp.logical_and(riota >= start, riota < end)
            v = _treemax(jnp.where(m, z, -jnp.inf))
            cur2 = out_ref[pl.ds(s, 1), :]
            out_ref[pl.ds(s, 1), :] = jnp.maximum(cur2, v)

    @pl.when(i == nblk - 1)
    def _finish():
        acc = out_ref[...]
        res = jnp.maximum(acc + b_ref[...], 0.0)
        out_ref[...] = jnp.where(acc == -jnp.inf, acc, res)


def kernel(x, pos, W, b, batch):
    n = x.shape[0]
    nblk = n // BLK

    w1 = W[:, :61].T
    w2 = W[:, 61:].T
    b2 = b.reshape(1, 64)
    batch = batch.astype(jnp.int32)
    batchv = batch.reshape(n // 128, 128)
    blo = batch[::BLK]
    bhi = batch[BLK - 1::BLK]

    grid_spec = pltpu.PrefetchScalarGridSpec(
        num_scalar_prefetch=2,
        grid=(nblk,),
        in_specs=[
            pl.BlockSpec((BLK, 61), lambda i, *_: (i, 0)),
            pl.BlockSpec((BLK, 3), lambda i, *_: (i, 0)),
            pl.BlockSpec((61, 64), lambda i, *_: (0, 0)),
            pl.BlockSpec((3, 64), lambda i, *_: (0, 0)),
            pl.BlockSpec((1, 64), lambda i, *_: (0, 0)),
            pl.BlockSpec((BPR, 128), lambda i, *_: (i, 0)),
        ],
        out_specs=pl.BlockSpec((NSEG, 64), lambda i, *_: (0, 0)),
    )

    return pl.pallas_call(
        _pool_kernel,
        grid_spec=grid_spec,
        out_shape=jax.ShapeDtypeStruct((NSEG, 64), jnp.float32),
    )(blo, bhi, x, pos, w1, w2, b2, batchv)
